# Initial kernel scaffold; baseline (speedup 1.0000x reference)
import jax
import jax.numpy as jnp
from jax.experimental import pallas as pl


def kernel(x, edge_index, W1, b1, W2, b2):
    raise NotImplementedError("write your pallas kernel here")


# speedup vs baseline: 20.8320x; 20.8320x over previous
import functools

import jax
import jax.numpy as jnp
from jax import lax
from jax.experimental import pallas as pl
from jax.experimental.pallas import tpu as pltpu
from jax.experimental.pallas import tpu_sc as plsc

N = 10000
E = 320000
D_IN = 128
D_HID = 16
N_CLS = 40

NSC = 2
NTILE = 16
NW = NSC * NTILE

N_PAD = 10240
ROWS_PER_TILE = N_PAD // NTILE
E_PAD = 4096 * 80
ROWS_E = E_PAD // 128
ROWS_E_TILE = ROWS_E // NW
D2 = 48

_mesh = plsc.VectorSubcoreMesh(core_axis_name="c", subcore_axis_name="s")
_sc_params = pltpu.CompilerParams(use_tc_tiling_on_sc=False)


def _zero_rows(buf, nrows, ncol16):
    def body(i, carry):
        for j in range(ncol16):
            buf[i, pl.ds(j * 16, 16)] = jnp.zeros((16,), jnp.float32)
        return carry
    lax.fori_loop(0, nrows, body, 0)


@functools.partial(
    pl.kernel,
    out_type=jax.ShapeDtypeStruct((NSC * N_PAD,), jnp.float32),
    mesh=_mesh,
    compiler_params=_sc_params,
    scratch_types=[
        pltpu.VMEM((ROWS_E_TILE, 128), jnp.int32),
        pltpu.VMEM((128,), jnp.float32),
        pltpu.VMEM((ROWS_PER_TILE,), jnp.float32),
        pltpu.VMEM_SHARED((N_PAD,), jnp.float32),
    ],
)
def _deg_kernel(dst_hbm, out_hbm, dst_v, ones_v, buf_v, deg_sh):
    cid = lax.axis_index("c")
    sid = lax.axis_index("s")
    wid = cid * NTILE + sid
    r0 = sid * ROWS_PER_TILE

    def fill_ones(i, carry):
        ones_v[pl.ds(i * 16, 16)] = jnp.full((16,), 1.0, jnp.float32)
        return carry
    lax.fori_loop(0, 128 // 16, fill_ones, 0)

    def fill_zero(i, carry):
        buf_v[pl.ds(i * 16, 16)] = jnp.zeros((16,), jnp.float32)
        return carry
    lax.fori_loop(0, ROWS_PER_TILE // 16, fill_zero, 0)

    pltpu.sync_copy(buf_v, deg_sh.at[pl.ds(r0, ROWS_PER_TILE)])
    pltpu.sync_copy(dst_hbm.at[pl.ds(wid * ROWS_E_TILE, ROWS_E_TILE)], dst_v)
    plsc.subcore_barrier()

    def step(c, carry):
        pltpu.sync_copy(ones_v, deg_sh.at[dst_v.at[c]], add=True)
        return carry
    lax.fori_loop(0, ROWS_E_TILE, step, 0)

    plsc.subcore_barrier()
    pltpu.sync_copy(deg_sh.at[pl.ds(r0, ROWS_PER_TILE)], buf_v)
    pltpu.sync_copy(buf_v, out_hbm.at[pl.ds(cid * N_PAD + r0, ROWS_PER_TILE)])


def _make_prop_kernel(d):
    G = 8
    ncol16 = d // 16

    @functools.partial(
        pl.kernel,
        out_type=jax.ShapeDtypeStruct((NSC * N_PAD, d), jnp.float32),
        mesh=_mesh,
        compiler_params=_sc_params,
        scratch_types=[
            pltpu.VMEM((ROWS_E_TILE, 128), jnp.int32),
            pltpu.VMEM((ROWS_E_TILE, 128), jnp.int32),
            pltpu.VMEM((G * 128, d), jnp.float32),
            pltpu.VMEM_SHARED((N_PAD, d), jnp.float32),
            pltpu.SemaphoreType.DMA,
        ],
    )
    def prop(g_hbm, src_hbm, dst_hbm, out_hbm, src_v, dst_v, rows_v, acc_sh, sem):
        cid = lax.axis_index("c")
        sid = lax.axis_index("s")
        wid = cid * NTILE + sid
        r0 = sid * ROWS_PER_TILE

        _zero_rows(rows_v, ROWS_PER_TILE, ncol16)
        pltpu.sync_copy(rows_v.at[pl.ds(0, ROWS_PER_TILE)],
                        acc_sh.at[pl.ds(r0, ROWS_PER_TILE)])
        pltpu.sync_copy(src_hbm.at[pl.ds(wid * ROWS_E_TILE, ROWS_E_TILE)], src_v)
        pltpu.sync_copy(dst_hbm.at[pl.ds(wid * ROWS_E_TILE, ROWS_E_TILE)], dst_v)
        plsc.subcore_barrier()

        def step(c, carry):
            cps = [
                pltpu.async_copy(g_hbm.at[src_v.at[c * G + j]],
                                 rows_v.at[pl.ds(j * 128, 128)], sem)
                for j in range(G)
            ]
            for cp in cps:
                cp.wait()
            for j in range(G):
                pltpu.sync_copy(rows_v.at[pl.ds(j * 128, 128)],
                                acc_sh.at[dst_v.at[c * G + j]], add=True)
            return carry
        lax.fori_loop(0, ROWS_E_TILE // G, step, 0)

        plsc.subcore_barrier()
        pltpu.sync_copy(acc_sh.at[pl.ds(r0, ROWS_PER_TILE)],
                        rows_v.at[pl.ds(0, ROWS_PER_TILE)])
        pltpu.sync_copy(rows_v.at[pl.ds(0, ROWS_PER_TILE)],
                        out_hbm.at[pl.ds(cid * N_PAD + r0, ROWS_PER_TILE)])

    return prop


_prop16 = _make_prop_kernel(D_HID)
_prop48 = _make_prop_kernel(D2)


_BLK = 1024
_GRID = N_PAD // _BLK


def _tc1_body(x_ref, w1_ref, dega_ref, degb_ref, g1_ref, self1_ref,
              dis_ref, inv_ref):
    h = jnp.dot(x_ref[...], w1_ref[...], preferred_element_type=jnp.float32)
    deg = dega_ref[...] + degb_ref[...] + 1.0
    dis = lax.rsqrt(deg)
    inv = 1.0 / deg
    g1_ref[...] = h * dis
    self1_ref[...] = h * inv
    dis_ref[...] = dis
    inv_ref[...] = inv


_tc1 = pl.pallas_call(
    _tc1_body,
    grid=(_GRID,),
    in_specs=[
        pl.BlockSpec((_BLK, D_IN), lambda i: (i, 0)),
        pl.BlockSpec((D_IN, D_HID), lambda i: (0, 0)),
        pl.BlockSpec((_BLK, 1), lambda i: (i, 0)),
        pl.BlockSpec((_BLK, 1), lambda i: (i, 0)),
    ],
    out_specs=[
        pl.BlockSpec((_BLK, D_HID), lambda i: (i, 0)),
        pl.BlockSpec((_BLK, D_HID), lambda i: (i, 0)),
        pl.BlockSpec((_BLK, 1), lambda i: (i, 0)),
        pl.BlockSpec((_BLK, 1), lambda i: (i, 0)),
    ],
    out_shape=[
        jax.ShapeDtypeStruct((N_PAD, D_HID), jnp.float32),
        jax.ShapeDtypeStruct((N_PAD, D_HID), jnp.float32),
        jax.ShapeDtypeStruct((N_PAD, 1), jnp.float32),
        jax.ShapeDtypeStruct((N_PAD, 1), jnp.float32),
    ],
)


def _tc2_body(acca_ref, accb_ref, self1_ref, dis_ref, inv_ref, w2_ref,
              b1_ref, g2_ref, self2_ref):
    out1 = dis_ref[...] * (acca_ref[...] + accb_ref[...]) + self1_ref[...] \
        + b1_ref[...]
    z = jnp.maximum(out1, 0.0)
    h2 = jnp.dot(z, w2_ref[...], preferred_element_type=jnp.float32)
    g2 = h2 * dis_ref[...]
    g2_ref[...] = jnp.concatenate(
        [g2, jnp.zeros((g2.shape[0], D2 - N_CLS), jnp.float32)], axis=1)
    self2_ref[...] = h2 * inv_ref[...]


_tc2 = pl.pallas_call(
    _tc2_body,
    grid=(_GRID,),
    in_specs=[
        pl.BlockSpec((_BLK, D_HID), lambda i: (i, 0)),
        pl.BlockSpec((_BLK, D_HID), lambda i: (i, 0)),
        pl.BlockSpec((_BLK, D_HID), lambda i: (i, 0)),
        pl.BlockSpec((_BLK, 1), lambda i: (i, 0)),
        pl.BlockSpec((_BLK, 1), lambda i: (i, 0)),
        pl.BlockSpec((D_HID, N_CLS), lambda i: (0, 0)),
        pl.BlockSpec((1, D_HID), lambda i: (0, 0)),
    ],
    out_specs=[
        pl.BlockSpec((_BLK, D2), lambda i: (i, 0)),
        pl.BlockSpec((_BLK, N_CLS), lambda i: (i, 0)),
    ],
    out_shape=[
        jax.ShapeDtypeStruct((N_PAD, D2), jnp.float32),
        jax.ShapeDtypeStruct((N_PAD, N_CLS), jnp.float32),
    ],
)


def _tc3_body(acca_ref, accb_ref, self2_ref, dis_ref, b2_ref, out_ref):
    s = dis_ref[...] * (acca_ref[...] + accb_ref[...])[:, :N_CLS] \
        + self2_ref[...] + b2_ref[...]
    m = jnp.max(s, axis=1, keepdims=True)
    e = s - m
    out_ref[...] = e - jnp.log(jnp.sum(jnp.exp(e), axis=1, keepdims=True))


_tc3 = pl.pallas_call(
    _tc3_body,
    grid=(_GRID,),
    in_specs=[
        pl.BlockSpec((_BLK, D2), lambda i: (i, 0)),
        pl.BlockSpec((_BLK, D2), lambda i: (i, 0)),
        pl.BlockSpec((_BLK, N_CLS), lambda i: (i, 0)),
        pl.BlockSpec((_BLK, 1), lambda i: (i, 0)),
        pl.BlockSpec((1, N_CLS), lambda i: (0, 0)),
    ],
    out_specs=pl.BlockSpec((_BLK, N_CLS), lambda i: (i, 0)),
    out_shape=jax.ShapeDtypeStruct((N_PAD, N_CLS), jnp.float32),
)


def kernel(x, edge_index, W1, b1, W2, b2):
    pad_idx = jnp.full((E_PAD - E,), N_PAD - 1, jnp.int32)
    src_p = jnp.concatenate([edge_index[0], pad_idx]).reshape(ROWS_E, 128)
    dst_p = jnp.concatenate([edge_index[1], pad_idx]).reshape(ROWS_E, 128)
    x_p = jnp.pad(x, ((0, N_PAD - N), (0, 0)))

    deg = _deg_kernel(dst_p).reshape(NSC, N_PAD)
    g1, self1, dis, inv = _tc1(x_p, W1,
                               deg[0].reshape(N_PAD, 1),
                               deg[1].reshape(N_PAD, 1))
    acc1 = _prop16(g1, src_p, dst_p).reshape(NSC, N_PAD, D_HID)
    g2, self2 = _tc2(acc1[0], acc1[1], self1, dis, inv, W2,
                     b1.reshape(1, D_HID))
    acc2 = _prop48(g2, src_p, dst_p).reshape(NSC, N_PAD, D2)
    out = _tc3(acc2[0], acc2[1], self2, dis, b2.reshape(1, N_CLS))
    return out[:N]


# speedup vs baseline: 22.0522x; 1.0586x over previous
import functools

import jax
import jax.numpy as jnp
from jax import lax
from jax.experimental import pallas as pl
from jax.experimental.pallas import tpu as pltpu
from jax.experimental.pallas import tpu_sc as plsc

N = 10000
E = 320000
D_IN = 128
D_HID = 16
N_CLS = 40

NSC = 2
NTILE = 16
NW = NSC * NTILE

N_PAD = 10240
ROWS_PER_TILE = N_PAD // NTILE
E_PAD = 4096 * 80
ROWS_E = E_PAD // 128
ROWS_E_TILE = ROWS_E // NW
D2 = 48

_mesh = plsc.VectorSubcoreMesh(core_axis_name="c", subcore_axis_name="s")
_sc_params = pltpu.CompilerParams(use_tc_tiling_on_sc=False)


def _zero_rows(buf, nrows, ncol16):
    def body(i, carry):
        for j in range(ncol16):
            buf[i, pl.ds(j * 16, 16)] = jnp.zeros((16,), jnp.float32)
        return carry
    lax.fori_loop(0, nrows, body, 0)


@functools.partial(
    pl.kernel,
    out_type=jax.ShapeDtypeStruct((NSC * N_PAD,), jnp.float32),
    mesh=_mesh,
    compiler_params=_sc_params,
    scratch_types=[
        pltpu.VMEM((ROWS_E_TILE, 128), jnp.int32),
        pltpu.VMEM((128,), jnp.float32),
        pltpu.VMEM((ROWS_PER_TILE,), jnp.float32),
        pltpu.VMEM_SHARED((N_PAD,), jnp.float32),
    ],
)
def _deg_kernel(dst_hbm, out_hbm, dst_v, ones_v, buf_v, deg_sh):
    cid = lax.axis_index("c")
    sid = lax.axis_index("s")
    wid = cid * NTILE + sid
    r0 = sid * ROWS_PER_TILE

    def fill_ones(i, carry):
        ones_v[pl.ds(i * 16, 16)] = jnp.full((16,), 1.0, jnp.float32)
        return carry
    lax.fori_loop(0, 128 // 16, fill_ones, 0)

    def fill_zero(i, carry):
        buf_v[pl.ds(i * 16, 16)] = jnp.zeros((16,), jnp.float32)
        return carry
    lax.fori_loop(0, ROWS_PER_TILE // 16, fill_zero, 0)

    pltpu.sync_copy(buf_v, deg_sh.at[pl.ds(r0, ROWS_PER_TILE)])
    pltpu.sync_copy(dst_hbm.at[pl.ds(wid * ROWS_E_TILE, ROWS_E_TILE)], dst_v)
    plsc.subcore_barrier()

    def step(c, carry):
        pltpu.sync_copy(ones_v, deg_sh.at[dst_v.at[c]], add=True)
        return carry
    lax.fori_loop(0, ROWS_E_TILE, step, 0)

    plsc.subcore_barrier()
    pltpu.sync_copy(deg_sh.at[pl.ds(r0, ROWS_PER_TILE)], buf_v)
    pltpu.sync_copy(buf_v, out_hbm.at[pl.ds(cid * N_PAD + r0, ROWS_PER_TILE)])


def _make_prop_kernel(d):
    G = 8 if d <= 16 else 4
    G128 = G * 128
    STEPS = ROWS_E_TILE // G
    PAIRS = STEPS // 2
    ncol16 = d // 16

    @functools.partial(
        pl.kernel,
        out_type=jax.ShapeDtypeStruct((NSC * N_PAD, d), jnp.float32),
        mesh=_mesh,
        compiler_params=_sc_params,
        scratch_types=[
            pltpu.VMEM((ROWS_E_TILE, 128), jnp.int32),
            pltpu.VMEM((ROWS_E_TILE, 128), jnp.int32),
            pltpu.VMEM((2 * G128, d), jnp.float32),
            pltpu.VMEM_SHARED((N_PAD, d), jnp.float32),
            pltpu.SemaphoreType.DMA,
            pltpu.SemaphoreType.DMA,
            pltpu.SemaphoreType.DMA,
        ],
    )
    def prop(g_hbm, src_hbm, dst_hbm, out_hbm, src_v, dst_v, rows_v, acc_sh,
             gsem0, gsem1, ssem):
        cid = lax.axis_index("c")
        sid = lax.axis_index("s")
        wid = cid * NTILE + sid
        r0 = sid * ROWS_PER_TILE

        _zero_rows(rows_v, ROWS_PER_TILE, ncol16)
        pltpu.sync_copy(rows_v.at[pl.ds(0, ROWS_PER_TILE)],
                        acc_sh.at[pl.ds(r0, ROWS_PER_TILE)])
        pltpu.sync_copy(src_hbm.at[pl.ds(wid * ROWS_E_TILE, ROWS_E_TILE)], src_v)
        pltpu.sync_copy(dst_hbm.at[pl.ds(wid * ROWS_E_TILE, ROWS_E_TILE)], dst_v)
        plsc.subcore_barrier()

        def fire_gathers(grp, off, sem):
            for j in range(G):
                pltpu.async_copy(g_hbm.at[src_v.at[grp * G + j]],
                                 rows_v.at[pl.ds(off + j * 128, 128)], sem)

        def drain_gathers(grp, off, sem):
            for j in range(G):
                pltpu.make_async_copy(
                    g_hbm.at[src_v.at[grp * G + j]],
                    rows_v.at[pl.ds(off + j * 128, 128)], sem).wait()

        def scatter_group(grp, off):
            cps = [
                pltpu.async_copy(rows_v.at[pl.ds(off + j * 128, 128)],
                                 acc_sh.at[dst_v.at[grp * G + j]], ssem,
                                 add=True)
                for j in range(G)
            ]
            for cp in cps:
                cp.wait()

        fire_gathers(0, 0, gsem0)

        def step(t, carry):
            fire_gathers(2 * t + 1, G128, gsem1)
            drain_gathers(2 * t, 0, gsem0)
            scatter_group(2 * t, 0)

            @pl.when(t + 1 < PAIRS)
            def _():
                fire_gathers(2 * t + 2, 0, gsem0)
            drain_gathers(2 * t + 1, G128, gsem1)
            scatter_group(2 * t + 1, G128)
            return carry
        lax.fori_loop(0, PAIRS, step, 0)

        plsc.subcore_barrier()
        pltpu.sync_copy(acc_sh.at[pl.ds(r0, ROWS_PER_TILE)],
                        rows_v.at[pl.ds(0, ROWS_PER_TILE)])
        pltpu.sync_copy(rows_v.at[pl.ds(0, ROWS_PER_TILE)],
                        out_hbm.at[pl.ds(cid * N_PAD + r0, ROWS_PER_TILE)])

    return prop


_prop16 = _make_prop_kernel(D_HID)
_prop48 = _make_prop_kernel(D2)


_BLK = 1024
_GRID = N_PAD // _BLK


def _tc1_body(x_ref, w1_ref, dega_ref, degb_ref, g1_ref, self1_ref,
              dis_ref, inv_ref):
    h = jnp.dot(x_ref[...], w1_ref[...], preferred_element_type=jnp.float32)
    deg = dega_ref[...] + degb_ref[...] + 1.0
    dis = lax.rsqrt(deg)
    inv = 1.0 / deg
    g1_ref[...] = h * dis
    self1_ref[...] = h * inv
    dis_ref[...] = dis
    inv_ref[...] = inv


_tc1 = pl.pallas_call(
    _tc1_body,
    grid=(_GRID,),
    in_specs=[
        pl.BlockSpec((_BLK, D_IN), lambda i: (i, 0)),
        pl.BlockSpec((D_IN, D_HID), lambda i: (0, 0)),
        pl.BlockSpec((_BLK, 1), lambda i: (i, 0)),
        pl.BlockSpec((_BLK, 1), lambda i: (i, 0)),
    ],
    out_specs=[
        pl.BlockSpec((_BLK, D_HID), lambda i: (i, 0)),
        pl.BlockSpec((_BLK, D_HID), lambda i: (i, 0)),
        pl.BlockSpec((_BLK, 1), lambda i: (i, 0)),
        pl.BlockSpec((_BLK, 1), lambda i: (i, 0)),
    ],
    out_shape=[
        jax.ShapeDtypeStruct((N_PAD, D_HID), jnp.float32),
        jax.ShapeDtypeStruct((N_PAD, D_HID), jnp.float32),
        jax.ShapeDtypeStruct((N_PAD, 1), jnp.float32),
        jax.ShapeDtypeStruct((N_PAD, 1), jnp.float32),
    ],
)


def _tc2_body(acca_ref, accb_ref, self1_ref, dis_ref, inv_ref, w2_ref,
              b1_ref, g2_ref, self2_ref):
    out1 = dis_ref[...] * (acca_ref[...] + accb_ref[...]) + self1_ref[...] \
        + b1_ref[...]
    z = jnp.maximum(out1, 0.0)
    h2 = jnp.dot(z, w2_ref[...], preferred_element_type=jnp.float32)
    g2 = h2 * dis_ref[...]
    g2_ref[...] = jnp.concatenate(
        [g2, jnp.zeros((g2.shape[0], D2 - N_CLS), jnp.float32)], axis=1)
    self2_ref[...] = h2 * inv_ref[...]


_tc2 = pl.pallas_call(
    _tc2_body,
    grid=(_GRID,),
    in_specs=[
        pl.BlockSpec((_BLK, D_HID), lambda i: (i, 0)),
        pl.BlockSpec((_BLK, D_HID), lambda i: (i, 0)),
        pl.BlockSpec((_BLK, D_HID), lambda i: (i, 0)),
        pl.BlockSpec((_BLK, 1), lambda i: (i, 0)),
        pl.BlockSpec((_BLK, 1), lambda i: (i, 0)),
        pl.BlockSpec((D_HID, N_CLS), lambda i: (0, 0)),
        pl.BlockSpec((1, D_HID), lambda i: (0, 0)),
    ],
    out_specs=[
        pl.BlockSpec((_BLK, D2), lambda i: (i, 0)),
        pl.BlockSpec((_BLK, N_CLS), lambda i: (i, 0)),
    ],
    out_shape=[
        jax.ShapeDtypeStruct((N_PAD, D2), jnp.float32),
        jax.ShapeDtypeStruct((N_PAD, N_CLS), jnp.float32),
    ],
)


def _tc3_body(acca_ref, accb_ref, self2_ref, dis_ref, b2_ref, out_ref):
    s = dis_ref[...] * (acca_ref[...] + accb_ref[...])[:, :N_CLS] \
        + self2_ref[...] + b2_ref[...]
    m = jnp.max(s, axis=1, keepdims=True)
    e = s - m
    out_ref[...] = e - jnp.log(jnp.sum(jnp.exp(e), axis=1, keepdims=True))


_tc3 = pl.pallas_call(
    _tc3_body,
    grid=(_GRID,),
    in_specs=[
        pl.BlockSpec((_BLK, D2), lambda i: (i, 0)),
        pl.BlockSpec((_BLK, D2), lambda i: (i, 0)),
        pl.BlockSpec((_BLK, N_CLS), lambda i: (i, 0)),
        pl.BlockSpec((_BLK, 1), lambda i: (i, 0)),
        pl.BlockSpec((1, N_CLS), lambda i: (0, 0)),
    ],
    out_specs=pl.BlockSpec((_BLK, N_CLS), lambda i: (i, 0)),
    out_shape=jax.ShapeDtypeStruct((N_PAD, N_CLS), jnp.float32),
)


def kernel(x, edge_index, W1, b1, W2, b2):
    pad_idx = jnp.full((E_PAD - E,), N_PAD - 1, jnp.int32)
    src_p = jnp.concatenate([edge_index[0], pad_idx]).reshape(ROWS_E, 128)
    dst_p = jnp.concatenate([edge_index[1], pad_idx]).reshape(ROWS_E, 128)
    x_p = jnp.pad(x, ((0, N_PAD - N), (0, 0)))

    deg = _deg_kernel(dst_p).reshape(NSC, N_PAD)
    g1, self1, dis, inv = _tc1(x_p, W1,
                               deg[0].reshape(N_PAD, 1),
                               deg[1].reshape(N_PAD, 1))
    acc1 = _prop16(g1, src_p, dst_p).reshape(NSC, N_PAD, D_HID)
    g2, self2 = _tc2(acc1[0], acc1[1], self1, dis, inv, W2,
                     b1.reshape(1, D_HID))
    acc2 = _prop48(g2, src_p, dst_p).reshape(NSC, N_PAD, D2)
    out = _tc3(acc2[0], acc2[1], self2, dis, b2.reshape(1, N_CLS))
    return out[:N]


# speedup vs baseline: 42.9338x; 1.9469x over previous
import functools

import jax
import jax.numpy as jnp
from jax import lax
from jax.experimental import pallas as pl
from jax.experimental.pallas import tpu as pltpu
from jax.experimental.pallas import tpu_sc as plsc

N = 10000
E = 320000
D_IN = 128
D_HID = 16
N_CLS = 40

NSC = 2
NTILE = 16
NW = NSC * NTILE

N_PAD = 10240
ROWS_PER_TILE = N_PAD // NTILE
E_PAD = 4096 * 80
ROWS_E = E_PAD // 128
ROWS_E_TILE = ROWS_E // NW
D2 = 48

_mesh = plsc.VectorSubcoreMesh(core_axis_name="c", subcore_axis_name="s")
_sc_params = pltpu.CompilerParams(use_tc_tiling_on_sc=False)


@functools.partial(
    pl.kernel,
    out_type=(jax.ShapeDtypeStruct((N_PAD,), jnp.float32),
              jax.ShapeDtypeStruct((N_PAD,), jnp.float32)),
    mesh=_mesh,
    compiler_params=_sc_params,
    scratch_types=[
        pltpu.VMEM((ROWS_E_TILE, 128), jnp.int32),
        pltpu.VMEM((128,), jnp.float32),
        pltpu.VMEM((ROWS_PER_TILE,), jnp.float32),
        pltpu.VMEM_SHARED((N_PAD,), jnp.float32),
    ],
)
def _deg_kernel(dst_hbm, out0, out1, dst_v, ones_v, buf_v, deg_sh):
    cid = lax.axis_index("c")
    sid = lax.axis_index("s")
    wid = cid * NTILE + sid
    r0 = sid * ROWS_PER_TILE

    def fill_ones(i, carry):
        ones_v[pl.ds(i * 16, 16)] = jnp.full((16,), 1.0, jnp.float32)
        return carry
    lax.fori_loop(0, 128 // 16, fill_ones, 0)

    def fill_zero(i, carry):
        buf_v[pl.ds(i * 16, 16)] = jnp.zeros((16,), jnp.float32)
        return carry
    lax.fori_loop(0, ROWS_PER_TILE // 16, fill_zero, 0)

    pltpu.sync_copy(buf_v, deg_sh.at[pl.ds(r0, ROWS_PER_TILE)])
    pltpu.sync_copy(dst_hbm.at[pl.ds(wid * ROWS_E_TILE, ROWS_E_TILE)], dst_v)
    plsc.subcore_barrier()

    def step(c, carry):
        pltpu.sync_copy(ones_v, deg_sh.at[dst_v.at[c]], add=True)
        return carry
    lax.fori_loop(0, ROWS_E_TILE, step, 0)

    plsc.subcore_barrier()
    pltpu.sync_copy(deg_sh.at[pl.ds(r0, ROWS_PER_TILE)], buf_v)

    @pl.when(cid == 0)
    def _():
        pltpu.sync_copy(buf_v, out0.at[pl.ds(r0, ROWS_PER_TILE)])

    @pl.when(cid == 1)
    def _():
        pltpu.sync_copy(buf_v, out1.at[pl.ds(r0, ROWS_PER_TILE)])


def _make_prop_kernel(d, g):
    G = g
    G128 = G * 128
    STEPS = ROWS_E_TILE // G
    PAIRS = STEPS // 2
    assert STEPS * G == ROWS_E_TILE and PAIRS * 2 == STEPS
    ZR = 128

    @functools.partial(
        pl.kernel,
        out_type=(jax.ShapeDtypeStruct((N_PAD, d), jnp.float32),
                  jax.ShapeDtypeStruct((N_PAD, d), jnp.float32)),
        mesh=_mesh,
        compiler_params=_sc_params,
        scratch_types=[
            pltpu.VMEM((ROWS_E_TILE, 128), jnp.int32),
            pltpu.VMEM((ROWS_E_TILE, 128), jnp.int32),
            pltpu.VMEM((2 * G128, d), jnp.float32),
            pltpu.VMEM((ZR, d), jnp.float32),
            pltpu.VMEM_SHARED((N_PAD, d), jnp.float32),
            pltpu.VMEM_SHARED((N_PAD, d), jnp.float32),
            pltpu.SemaphoreType.DMA,
            pltpu.SemaphoreType.DMA,
            pltpu.SemaphoreType.DMA,
        ],
    )
    def prop(g_hbm, src_hbm, dst_hbm, out0, out1, src_v, dst_v, rows_v,
             zero_v, table_sh, acc_sh, gsem0, gsem1, ssem):
        cid = lax.axis_index("c")
        sid = lax.axis_index("s")
        wid = cid * NTILE + sid
        r0 = sid * ROWS_PER_TILE

        def zrow(i, carry):
            for j in range(d // 16):
                zero_v[i, pl.ds(j * 16, 16)] = jnp.zeros((16,), jnp.float32)
            return carry
        lax.fori_loop(0, ZR, zrow, 0)
        for z in range(ROWS_PER_TILE // ZR):
            pltpu.sync_copy(zero_v, acc_sh.at[pl.ds(r0 + z * ZR, ZR)])
        pltpu.sync_copy(g_hbm.at[pl.ds(r0, ROWS_PER_TILE)],
                        table_sh.at[pl.ds(r0, ROWS_PER_TILE)])
        pltpu.sync_copy(src_hbm.at[pl.ds(wid * ROWS_E_TILE, ROWS_E_TILE)], src_v)
        pltpu.sync_copy(dst_hbm.at[pl.ds(wid * ROWS_E_TILE, ROWS_E_TILE)], dst_v)
        plsc.subcore_barrier()

        def fire_gathers(grp, off, sem):
            for j in range(G):
                pltpu.async_copy(table_sh.at[src_v.at[grp * G + j]],
                                 rows_v.at[pl.ds(off + j * 128, 128)], sem)

        def drain_gathers(grp, off, sem):
            for j in range(G):
                pltpu.make_async_copy(
                    table_sh.at[src_v.at[grp * G + j]],
                    rows_v.at[pl.ds(off + j * 128, 128)], sem).wait()

        def scatter_group(grp, off):
            cps = [
                pltpu.async_copy(rows_v.at[pl.ds(off + j * 128, 128)],
                                 acc_sh.at[dst_v.at[grp * G + j]], ssem,
                                 add=True)
                for j in range(G)
            ]
            for cp in cps:
                cp.wait()

        fire_gathers(0, 0, gsem0)

        def step(t, carry):
            fire_gathers(2 * t + 1, G128, gsem1)
            drain_gathers(2 * t, 0, gsem0)
            scatter_group(2 * t, 0)

            @pl.when(t + 1 < PAIRS)
            def _():
                fire_gathers(2 * t + 2, 0, gsem0)
            drain_gathers(2 * t + 1, G128, gsem1)
            scatter_group(2 * t + 1, G128)
            return carry
        lax.fori_loop(0, PAIRS, step, 0)

        plsc.subcore_barrier()

        @pl.when(cid == 0)
        def _():
            pltpu.sync_copy(acc_sh.at[pl.ds(r0, ROWS_PER_TILE)],
                            out0.at[pl.ds(r0, ROWS_PER_TILE)])

        @pl.when(cid == 1)
        def _():
            pltpu.sync_copy(acc_sh.at[pl.ds(r0, ROWS_PER_TILE)],
                            out1.at[pl.ds(r0, ROWS_PER_TILE)])

    return prop


_prop16 = _make_prop_kernel(D_HID, 8)
_prop48 = _make_prop_kernel(D2, 2)


_BLK = 1024
_GRID = N_PAD // _BLK


def _tc1_body(x_ref, w1_ref, dega_ref, degb_ref, g1_ref, self1_ref,
              dis_ref, inv_ref):
    h = jnp.dot(x_ref[...], w1_ref[...], preferred_element_type=jnp.float32)
    deg = dega_ref[...] + degb_ref[...] + 1.0
    dis = lax.rsqrt(deg)
    inv = 1.0 / deg
    g1_ref[...] = h * dis
    self1_ref[...] = h * inv
    dis_ref[...] = dis
    inv_ref[...] = inv


_tc1 = pl.pallas_call(
    _tc1_body,
    grid=(_GRID,),
    in_specs=[
        pl.BlockSpec((_BLK, D_IN), lambda i: (i, 0)),
        pl.BlockSpec((D_IN, D_HID), lambda i: (0, 0)),
        pl.BlockSpec((_BLK, 1), lambda i: (i, 0)),
        pl.BlockSpec((_BLK, 1), lambda i: (i, 0)),
    ],
    out_specs=[
        pl.BlockSpec((_BLK, D_HID), lambda i: (i, 0)),
        pl.BlockSpec((_BLK, D_HID), lambda i: (i, 0)),
        pl.BlockSpec((_BLK, 1), lambda i: (i, 0)),
        pl.BlockSpec((_BLK, 1), lambda i: (i, 0)),
    ],
    out_shape=[
        jax.ShapeDtypeStruct((N_PAD, D_HID), jnp.float32),
        jax.ShapeDtypeStruct((N_PAD, D_HID), jnp.float32),
        jax.ShapeDtypeStruct((N_PAD, 1), jnp.float32),
        jax.ShapeDtypeStruct((N_PAD, 1), jnp.float32),
    ],
)


def _tc2_body(acca_ref, accb_ref, self1_ref, dis_ref, inv_ref, w2_ref,
              b1_ref, g2_ref, self2_ref):
    out1 = dis_ref[...] * (acca_ref[...] + accb_ref[...]) + self1_ref[...] \
        + b1_ref[...]
    z = jnp.maximum(out1, 0.0)
    h2 = jnp.dot(z, w2_ref[...], preferred_element_type=jnp.float32)
    g2 = h2 * dis_ref[...]
    g2_ref[...] = jnp.concatenate(
        [g2, jnp.zeros((g2.shape[0], D2 - N_CLS), jnp.float32)], axis=1)
    self2_ref[...] = h2 * inv_ref[...]


_tc2 = pl.pallas_call(
    _tc2_body,
    grid=(_GRID,),
    in_specs=[
        pl.BlockSpec((_BLK, D_HID), lambda i: (i, 0)),
        pl.BlockSpec((_BLK, D_HID), lambda i: (i, 0)),
        pl.BlockSpec((_BLK, D_HID), lambda i: (i, 0)),
        pl.BlockSpec((_BLK, 1), lambda i: (i, 0)),
        pl.BlockSpec((_BLK, 1), lambda i: (i, 0)),
        pl.BlockSpec((D_HID, N_CLS), lambda i: (0, 0)),
        pl.BlockSpec((1, D_HID), lambda i: (0, 0)),
    ],
    out_specs=[
        pl.BlockSpec((_BLK, D2), lambda i: (i, 0)),
        pl.BlockSpec((_BLK, N_CLS), lambda i: (i, 0)),
    ],
    out_shape=[
        jax.ShapeDtypeStruct((N_PAD, D2), jnp.float32),
        jax.ShapeDtypeStruct((N_PAD, N_CLS), jnp.float32),
    ],
)


def _tc3_body(acca_ref, accb_ref, self2_ref, dis_ref, b2_ref, out_ref):
    s = dis_ref[...] * (acca_ref[...] + accb_ref[...])[:, :N_CLS] \
        + self2_ref[...] + b2_ref[...]
    m = jnp.max(s, axis=1, keepdims=True)
    e = s - m
    out_ref[...] = e - jnp.log(jnp.sum(jnp.exp(e), axis=1, keepdims=True))


_tc3 = pl.pallas_call(
    _tc3_body,
    grid=(_GRID,),
    in_specs=[
        pl.BlockSpec((_BLK, D2), lambda i: (i, 0)),
        pl.BlockSpec((_BLK, D2), lambda i: (i, 0)),
        pl.BlockSpec((_BLK, N_CLS), lambda i: (i, 0)),
        pl.BlockSpec((_BLK, 1), lambda i: (i, 0)),
        pl.BlockSpec((1, N_CLS), lambda i: (0, 0)),
    ],
    out_specs=pl.BlockSpec((_BLK, N_CLS), lambda i: (i, 0)),
    out_shape=jax.ShapeDtypeStruct((N_PAD, N_CLS), jnp.float32),
)


def kernel(x, edge_index, W1, b1, W2, b2):
    pad_idx = jnp.full((E_PAD - E,), N_PAD - 1, jnp.int32)
    src_p = jnp.concatenate([edge_index[0], pad_idx]).reshape(ROWS_E, 128)
    dst_p = jnp.concatenate([edge_index[1], pad_idx]).reshape(ROWS_E, 128)
    x_p = jnp.pad(x, ((0, N_PAD - N), (0, 0)))

    deg_a, deg_b = _deg_kernel(dst_p)
    g1, self1, dis, inv = _tc1(x_p, W1,
                               deg_a.reshape(N_PAD, 1),
                               deg_b.reshape(N_PAD, 1))
    acc1_a, acc1_b = _prop16(g1, src_p, dst_p)
    g2, self2 = _tc2(acc1_a, acc1_b, self1, dis, inv, W2,
                     b1.reshape(1, D_HID))
    acc2_a, acc2_b = _prop48(g2, src_p, dst_p)
    out = _tc3(acc2_a, acc2_b, self2, dis, b2.reshape(1, N_CLS))
    return out[:N]


# speedup vs baseline: 46.6606x; 1.0868x over previous
import functools

import jax
import jax.numpy as jnp
from jax import lax
from jax.experimental import pallas as pl
from jax.experimental.pallas import tpu as pltpu
from jax.experimental.pallas import tpu_sc as plsc

N = 10000
E = 320000
D_IN = 128
D_HID = 16
N_CLS = 40

NSC = 2
NTILE = 16
NW = NSC * NTILE

N_PAD = 10240
ROWS_PER_TILE = N_PAD // NTILE
E_PAD = 4096 * 80
ROWS_E = E_PAD // 128
ROWS_E_TILE = ROWS_E // NW

_mesh = plsc.VectorSubcoreMesh(core_axis_name="c", subcore_axis_name="s")
_sc_params = pltpu.CompilerParams(use_tc_tiling_on_sc=False)


@functools.partial(
    pl.kernel,
    out_type=(jax.ShapeDtypeStruct((N_PAD,), jnp.float32),
              jax.ShapeDtypeStruct((N_PAD,), jnp.float32)),
    mesh=_mesh,
    compiler_params=_sc_params,
    scratch_types=[
        pltpu.VMEM((ROWS_E_TILE, 128), jnp.int32),
        pltpu.VMEM((128,), jnp.float32),
        pltpu.VMEM((ROWS_PER_TILE,), jnp.float32),
        pltpu.VMEM_SHARED((N_PAD,), jnp.float32),
    ],
)
def _deg_kernel(edges_hbm, out0, out1, dst_v, ones_v, buf_v, deg_sh):
    cid = lax.axis_index("c")
    sid = lax.axis_index("s")
    wid = cid * NTILE + sid
    r0 = sid * ROWS_PER_TILE

    def fill_ones(i, carry):
        ones_v[pl.ds(i * 16, 16)] = jnp.full((16,), 1.0, jnp.float32)
        return carry
    lax.fori_loop(0, 128 // 16, fill_ones, 0)

    def fill_zero(i, carry):
        buf_v[pl.ds(i * 16, 16)] = jnp.zeros((16,), jnp.float32)
        return carry
    lax.fori_loop(0, ROWS_PER_TILE // 16, fill_zero, 0)

    pltpu.sync_copy(buf_v, deg_sh.at[pl.ds(r0, ROWS_PER_TILE)])
    pltpu.sync_copy(
        edges_hbm.at[pl.ds(ROWS_E + wid * ROWS_E_TILE, ROWS_E_TILE)], dst_v)
    plsc.subcore_barrier()

    def step(c, carry):
        pltpu.sync_copy(ones_v, deg_sh.at[dst_v.at[c]], add=True)
        return carry
    lax.fori_loop(0, ROWS_E_TILE, step, 0)

    plsc.subcore_barrier()
    pltpu.sync_copy(deg_sh.at[pl.ds(r0, ROWS_PER_TILE)], buf_v)

    @pl.when(cid == 0)
    def _():
        pltpu.sync_copy(buf_v, out0.at[pl.ds(r0, ROWS_PER_TILE)])

    @pl.when(cid == 1)
    def _():
        pltpu.sync_copy(buf_v, out1.at[pl.ds(r0, ROWS_PER_TILE)])


def _make_prop_kernel(d, g):
    G = g
    G128 = G * 128
    STEPS = ROWS_E_TILE // G
    PAIRS = STEPS // 2
    assert STEPS * G == ROWS_E_TILE and PAIRS * 2 == STEPS
    ZR = 128

    @functools.partial(
        pl.kernel,
        out_type=(jax.ShapeDtypeStruct((N_PAD, d), jnp.float32),
                  jax.ShapeDtypeStruct((N_PAD, d), jnp.float32)),
        mesh=_mesh,
        compiler_params=_sc_params,
        scratch_types=[
            pltpu.VMEM((ROWS_E_TILE, 128), jnp.int32),
            pltpu.VMEM((ROWS_E_TILE, 128), jnp.int32),
            pltpu.VMEM((2 * G128, d), jnp.float32),
            pltpu.VMEM((ZR, d), jnp.float32),
            pltpu.VMEM_SHARED((N_PAD, d), jnp.float32),
            pltpu.VMEM_SHARED((N_PAD, d), jnp.float32),
            pltpu.SemaphoreType.DMA,
            pltpu.SemaphoreType.DMA,
            pltpu.SemaphoreType.DMA,
        ],
    )
    def prop(g_hbm, edges_hbm, out0, out1, src_v, dst_v, rows_v,
             zero_v, table_sh, acc_sh, gsem0, gsem1, ssem):
        cid = lax.axis_index("c")
        sid = lax.axis_index("s")
        wid = cid * NTILE + sid
        r0 = sid * ROWS_PER_TILE

        def zrow(i, carry):
            for j in range(d // 16):
                zero_v[i, pl.ds(j * 16, 16)] = jnp.zeros((16,), jnp.float32)
            if d % 16:
                zero_v[i, pl.ds(d - 16, 16)] = jnp.zeros((16,), jnp.float32)
            return carry
        lax.fori_loop(0, ZR, zrow, 0)
        for z in range(ROWS_PER_TILE // ZR):
            pltpu.sync_copy(zero_v, acc_sh.at[pl.ds(r0 + z * ZR, ZR)])
        pltpu.sync_copy(g_hbm.at[pl.ds(r0, ROWS_PER_TILE)],
                        table_sh.at[pl.ds(r0, ROWS_PER_TILE)])
        pltpu.sync_copy(
            edges_hbm.at[pl.ds(wid * ROWS_E_TILE, ROWS_E_TILE)], src_v)
        pltpu.sync_copy(
            edges_hbm.at[pl.ds(ROWS_E + wid * ROWS_E_TILE, ROWS_E_TILE)], dst_v)
        plsc.subcore_barrier()

        def fire_gathers(grp, off, sem):
            for j in range(G):
                pltpu.async_copy(table_sh.at[src_v.at[grp * G + j]],
                                 rows_v.at[pl.ds(off + j * 128, 128)], sem)

        def drain_gathers(grp, off, sem):
            for j in range(G):
                pltpu.make_async_copy(
                    table_sh.at[src_v.at[grp * G + j]],
                    rows_v.at[pl.ds(off + j * 128, 128)], sem).wait()

        def scatter_group(grp, off):
            cps = [
                pltpu.async_copy(rows_v.at[pl.ds(off + j * 128, 128)],
                                 acc_sh.at[dst_v.at[grp * G + j]], ssem,
                                 add=True)
                for j in range(G)
            ]
            for cp in cps:
                cp.wait()

        fire_gathers(0, 0, gsem0)

        def step(t, carry):
            fire_gathers(2 * t + 1, G128, gsem1)
            drain_gathers(2 * t, 0, gsem0)
            scatter_group(2 * t, 0)

            @pl.when(t + 1 < PAIRS)
            def _():
                fire_gathers(2 * t + 2, 0, gsem0)
            drain_gathers(2 * t + 1, G128, gsem1)
            scatter_group(2 * t + 1, G128)
            return carry
        lax.fori_loop(0, PAIRS, step, 0)

        plsc.subcore_barrier()

        @pl.when(cid == 0)
        def _():
            pltpu.sync_copy(acc_sh.at[pl.ds(r0, ROWS_PER_TILE)],
                            out0.at[pl.ds(r0, ROWS_PER_TILE)])

        @pl.when(cid == 1)
        def _():
            pltpu.sync_copy(acc_sh.at[pl.ds(r0, ROWS_PER_TILE)],
                            out1.at[pl.ds(r0, ROWS_PER_TILE)])

    return prop


_prop16 = _make_prop_kernel(D_HID, 8)
_prop40 = _make_prop_kernel(N_CLS, 2)


_BLK = 1000
_GRID = N // _BLK


def _mm1_body(x_ref, w1_ref, h_ref):
    h_ref[...] = jnp.dot(x_ref[...], w1_ref[...],
                         preferred_element_type=jnp.float32)


_mm1 = pl.pallas_call(
    _mm1_body,
    grid=(_GRID,),
    in_specs=[
        pl.BlockSpec((_BLK, D_IN), lambda i: (i, 0)),
        pl.BlockSpec((D_IN, D_HID), lambda i: (0, 0)),
    ],
    out_specs=pl.BlockSpec((_BLK, D_HID), lambda i: (i, 0)),
    out_shape=jax.ShapeDtypeStruct((N_PAD, D_HID), jnp.float32),
)


def _tc1_body(h_ref, dega_ref, degb_ref, g1_ref, self1_ref,
              dis_ref, inv_ref):
    h = h_ref[...]
    deg = dega_ref[...] + degb_ref[...] + 1.0
    dis = lax.rsqrt(deg)
    inv = 1.0 / deg
    g1_ref[...] = h * dis
    self1_ref[...] = h * inv
    dis_ref[...] = dis
    inv_ref[...] = inv


_tc1 = pl.pallas_call(
    _tc1_body,
    grid=(_GRID,),
    in_specs=[
        pl.BlockSpec((_BLK, D_HID), lambda i: (i, 0)),
        pl.BlockSpec((_BLK, 1), lambda i: (i, 0)),
        pl.BlockSpec((_BLK, 1), lambda i: (i, 0)),
    ],
    out_specs=[
        pl.BlockSpec((_BLK, D_HID), lambda i: (i, 0)),
        pl.BlockSpec((_BLK, D_HID), lambda i: (i, 0)),
        pl.BlockSpec((_BLK, 1), lambda i: (i, 0)),
        pl.BlockSpec((_BLK, 1), lambda i: (i, 0)),
    ],
    out_shape=[
        jax.ShapeDtypeStruct((N_PAD, D_HID), jnp.float32),
        jax.ShapeDtypeStruct((N_PAD, D_HID), jnp.float32),
        jax.ShapeDtypeStruct((N_PAD, 1), jnp.float32),
        jax.ShapeDtypeStruct((N_PAD, 1), jnp.float32),
    ],
)


def _tc2_body(acca_ref, accb_ref, self1_ref, dis_ref, inv_ref, w2_ref,
              b1_ref, g2_ref, self2_ref):
    out1 = dis_ref[...] * (acca_ref[...] + accb_ref[...]) + self1_ref[...] \
        + b1_ref[...]
    z = jnp.maximum(out1, 0.0)
    h2 = jnp.dot(z, w2_ref[...], preferred_element_type=jnp.float32)
    g2_ref[...] = h2 * dis_ref[...]
    self2_ref[...] = h2 * inv_ref[...]


_tc2 = pl.pallas_call(
    _tc2_body,
    grid=(_GRID,),
    in_specs=[
        pl.BlockSpec((_BLK, D_HID), lambda i: (i, 0)),
        pl.BlockSpec((_BLK, D_HID), lambda i: (i, 0)),
        pl.BlockSpec((_BLK, D_HID), lambda i: (i, 0)),
        pl.BlockSpec((_BLK, 1), lambda i: (i, 0)),
        pl.BlockSpec((_BLK, 1), lambda i: (i, 0)),
        pl.BlockSpec((D_HID, N_CLS), lambda i: (0, 0)),
        pl.BlockSpec((1, D_HID), lambda i: (0, 0)),
    ],
    out_specs=[
        pl.BlockSpec((_BLK, N_CLS), lambda i: (i, 0)),
        pl.BlockSpec((_BLK, N_CLS), lambda i: (i, 0)),
    ],
    out_shape=[
        jax.ShapeDtypeStruct((N_PAD, N_CLS), jnp.float32),
        jax.ShapeDtypeStruct((N_PAD, N_CLS), jnp.float32),
    ],
)


def _tc3_body(acca_ref, accb_ref, self2_ref, dis_ref, b2_ref, out_ref):
    s = dis_ref[...] * (acca_ref[...] + accb_ref[...]) \
        + self2_ref[...] + b2_ref[...]
    m = jnp.max(s, axis=1, keepdims=True)
    e = s - m
    out_ref[...] = e - jnp.log(jnp.sum(jnp.exp(e), axis=1, keepdims=True))


_tc3 = pl.pallas_call(
    _tc3_body,
    grid=(_GRID,),
    in_specs=[
        pl.BlockSpec((_BLK, N_CLS), lambda i: (i, 0)),
        pl.BlockSpec((_BLK, N_CLS), lambda i: (i, 0)),
        pl.BlockSpec((_BLK, N_CLS), lambda i: (i, 0)),
        pl.BlockSpec((_BLK, 1), lambda i: (i, 0)),
        pl.BlockSpec((1, N_CLS), lambda i: (0, 0)),
    ],
    out_specs=pl.BlockSpec((_BLK, N_CLS), lambda i: (i, 0)),
    out_shape=jax.ShapeDtypeStruct((N, N_CLS), jnp.float32),
)


def kernel(x, edge_index, W1, b1, W2, b2):
    edges = jnp.pad(edge_index, ((0, 0), (0, E_PAD - E)),
                    constant_values=N_PAD - 1).reshape(2 * ROWS_E, 128)

    h1 = _mm1(x, W1)
    deg_a, deg_b = _deg_kernel(edges)
    g1, self1, dis, inv = _tc1(h1,
                               deg_a.reshape(N_PAD, 1),
                               deg_b.reshape(N_PAD, 1))
    acc1_a, acc1_b = _prop16(g1, edges)
    g2, self2 = _tc2(acc1_a, acc1_b, self1, dis, inv, W2,
                     b1.reshape(1, D_HID))
    acc2_a, acc2_b = _prop40(g2, edges)
    return _tc3(acc2_a, acc2_b, self2, dis, b2.reshape(1, N_CLS))


# speedup vs baseline: 47.4022x; 1.0159x over previous
import functools

import jax
import jax.numpy as jnp
from jax import lax
from jax.experimental import pallas as pl
from jax.experimental.pallas import tpu as pltpu
from jax.experimental.pallas import tpu_sc as plsc

N = 10000
E = 320000
D_IN = 128
D_HID = 16
N_CLS = 40

NSC = 2
NTILE = 16
NW = NSC * NTILE

N_PAD = 10240
ROWS_PER_TILE = N_PAD // NTILE
E_PAD = 4096 * 80
ROWS_E = E_PAD // 128
ROWS_E_TILE = ROWS_E // NW

_mesh = plsc.VectorSubcoreMesh(core_axis_name="c", subcore_axis_name="s")
_sc_params = pltpu.CompilerParams(use_tc_tiling_on_sc=False)


@functools.partial(
    pl.kernel,
    out_type=(jax.ShapeDtypeStruct((N_PAD,), jnp.float32),
              jax.ShapeDtypeStruct((N_PAD,), jnp.float32)),
    mesh=_mesh,
    compiler_params=_sc_params,
    scratch_types=[
        pltpu.VMEM((ROWS_E_TILE, 128), jnp.int32),
        pltpu.VMEM((128,), jnp.float32),
        pltpu.VMEM((ROWS_PER_TILE,), jnp.float32),
        pltpu.VMEM_SHARED((N_PAD,), jnp.float32),
    ],
)
def _deg_kernel(edges_hbm, out0, out1, dst_v, ones_v, buf_v, deg_sh):
    cid = lax.axis_index("c")
    sid = lax.axis_index("s")
    wid = cid * NTILE + sid
    r0 = sid * ROWS_PER_TILE

    def fill_ones(i, carry):
        ones_v[pl.ds(i * 16, 16)] = jnp.full((16,), 1.0, jnp.float32)
        return carry
    lax.fori_loop(0, 128 // 16, fill_ones, 0)

    def fill_zero(i, carry):
        buf_v[pl.ds(i * 16, 16)] = jnp.zeros((16,), jnp.float32)
        return carry
    lax.fori_loop(0, ROWS_PER_TILE // 16, fill_zero, 0)

    pltpu.sync_copy(buf_v, deg_sh.at[pl.ds(r0, ROWS_PER_TILE)])
    pltpu.sync_copy(
        edges_hbm.at[pl.ds(ROWS_E + wid * ROWS_E_TILE, ROWS_E_TILE)], dst_v)
    plsc.subcore_barrier()

    def step(c, carry):
        pltpu.sync_copy(ones_v, deg_sh.at[dst_v.at[c]], add=True)
        return carry
    lax.fori_loop(0, ROWS_E_TILE, step, 0)

    plsc.subcore_barrier()
    pltpu.sync_copy(deg_sh.at[pl.ds(r0, ROWS_PER_TILE)], buf_v)

    @pl.when(cid == 0)
    def _():
        pltpu.sync_copy(buf_v, out0.at[pl.ds(r0, ROWS_PER_TILE)])

    @pl.when(cid == 1)
    def _():
        pltpu.sync_copy(buf_v, out1.at[pl.ds(r0, ROWS_PER_TILE)])


def _make_prop_kernel(d, g):
    G = g
    G128 = G * 128
    STEPS = ROWS_E_TILE // G
    PAIRS = STEPS // 2
    assert STEPS * G == ROWS_E_TILE and PAIRS * 2 == STEPS
    ZR = 128

    @functools.partial(
        pl.kernel,
        out_type=(jax.ShapeDtypeStruct((N_PAD, d), jnp.float32),
                  jax.ShapeDtypeStruct((N_PAD, d), jnp.float32)),
        mesh=_mesh,
        compiler_params=_sc_params,
        scratch_types=[
            pltpu.VMEM((ROWS_E_TILE, 128), jnp.int32),
            pltpu.VMEM((ROWS_E_TILE, 128), jnp.int32),
            pltpu.VMEM((2 * G128, d), jnp.float32),
            pltpu.VMEM((ZR, d), jnp.float32),
            pltpu.VMEM_SHARED((N_PAD, d), jnp.float32),
            pltpu.VMEM_SHARED((N_PAD, d), jnp.float32),
            pltpu.SemaphoreType.DMA,
            pltpu.SemaphoreType.DMA,
            pltpu.SemaphoreType.DMA,
        ],
    )
    def prop(g_hbm, edges_hbm, out0, out1, src_v, dst_v, rows_v,
             zero_v, table_sh, acc_sh, gsem0, gsem1, ssem):
        cid = lax.axis_index("c")
        sid = lax.axis_index("s")
        wid = cid * NTILE + sid
        r0 = sid * ROWS_PER_TILE

        def zrow(i, carry):
            for j in range(d // 16):
                zero_v[i, pl.ds(j * 16, 16)] = jnp.zeros((16,), jnp.float32)
            if d % 16:
                zero_v[i, pl.ds(d - 16, 16)] = jnp.zeros((16,), jnp.float32)
            return carry
        lax.fori_loop(0, ZR, zrow, 0)
        for z in range(ROWS_PER_TILE // ZR):
            pltpu.sync_copy(zero_v, acc_sh.at[pl.ds(r0 + z * ZR, ZR)])
        pltpu.sync_copy(g_hbm.at[pl.ds(r0, ROWS_PER_TILE)],
                        table_sh.at[pl.ds(r0, ROWS_PER_TILE)])
        pltpu.sync_copy(
            edges_hbm.at[pl.ds(wid * ROWS_E_TILE, ROWS_E_TILE)], src_v)
        pltpu.sync_copy(
            edges_hbm.at[pl.ds(ROWS_E + wid * ROWS_E_TILE, ROWS_E_TILE)], dst_v)
        plsc.subcore_barrier()

        def fire_gathers(grp, off, sem):
            for j in range(G):
                pltpu.async_copy(table_sh.at[src_v.at[grp * G + j]],
                                 rows_v.at[pl.ds(off + j * 128, 128)], sem)

        def drain_gathers(grp, off, sem):
            for j in range(G):
                pltpu.make_async_copy(
                    table_sh.at[src_v.at[grp * G + j]],
                    rows_v.at[pl.ds(off + j * 128, 128)], sem).wait()

        def scatter_group(grp, off):
            cps = [
                pltpu.async_copy(rows_v.at[pl.ds(off + j * 128, 128)],
                                 acc_sh.at[dst_v.at[grp * G + j]], ssem,
                                 add=True)
                for j in range(G)
            ]
            for cp in cps:
                cp.wait()

        fire_gathers(0, 0, gsem0)

        def step(t, carry):
            fire_gathers(2 * t + 1, G128, gsem1)
            drain_gathers(2 * t, 0, gsem0)
            scatter_group(2 * t, 0)

            @pl.when(t + 1 < PAIRS)
            def _():
                fire_gathers(2 * t + 2, 0, gsem0)
            drain_gathers(2 * t + 1, G128, gsem1)
            scatter_group(2 * t + 1, G128)
            return carry
        lax.fori_loop(0, PAIRS, step, 0)

        plsc.subcore_barrier()

        @pl.when(cid == 0)
        def _():
            pltpu.sync_copy(acc_sh.at[pl.ds(r0, ROWS_PER_TILE)],
                            out0.at[pl.ds(r0, ROWS_PER_TILE)])

        @pl.when(cid == 1)
        def _():
            pltpu.sync_copy(acc_sh.at[pl.ds(r0, ROWS_PER_TILE)],
                            out1.at[pl.ds(r0, ROWS_PER_TILE)])

    return prop


_prop16 = _make_prop_kernel(D_HID, 8)
_prop40 = _make_prop_kernel(N_CLS, 2)


_BLK = 10000
_GRID = N // _BLK


def _mm1_body(x_ref, w1_ref, h_ref):
    h_ref[...] = jnp.dot(x_ref[...], w1_ref[...],
                         preferred_element_type=jnp.float32)


_mm1 = pl.pallas_call(
    _mm1_body,
    grid=(_GRID,),
    in_specs=[
        pl.BlockSpec((_BLK, D_IN), lambda i: (i, 0)),
        pl.BlockSpec((D_IN, D_HID), lambda i: (0, 0)),
    ],
    out_specs=pl.BlockSpec((_BLK, D_HID), lambda i: (i, 0)),
    out_shape=jax.ShapeDtypeStruct((N_PAD, D_HID), jnp.float32),
)


def _tc1_body(h_ref, dega_ref, degb_ref, g1_ref, self1_ref,
              dis_ref, inv_ref):
    h = h_ref[...]
    deg = dega_ref[...] + degb_ref[...] + 1.0
    dis = lax.rsqrt(deg)
    inv = 1.0 / deg
    g1_ref[...] = h * dis
    self1_ref[...] = h * inv
    dis_ref[...] = dis
    inv_ref[...] = inv


_tc1 = pl.pallas_call(
    _tc1_body,
    grid=(_GRID,),
    in_specs=[
        pl.BlockSpec((_BLK, D_HID), lambda i: (i, 0)),
        pl.BlockSpec((_BLK, 1), lambda i: (i, 0)),
        pl.BlockSpec((_BLK, 1), lambda i: (i, 0)),
    ],
    out_specs=[
        pl.BlockSpec((_BLK, D_HID), lambda i: (i, 0)),
        pl.BlockSpec((_BLK, D_HID), lambda i: (i, 0)),
        pl.BlockSpec((_BLK, 1), lambda i: (i, 0)),
        pl.BlockSpec((_BLK, 1), lambda i: (i, 0)),
    ],
    out_shape=[
        jax.ShapeDtypeStruct((N_PAD, D_HID), jnp.float32),
        jax.ShapeDtypeStruct((N_PAD, D_HID), jnp.float32),
        jax.ShapeDtypeStruct((N_PAD, 1), jnp.float32),
        jax.ShapeDtypeStruct((N_PAD, 1), jnp.float32),
    ],
)


def _tc2_body(acca_ref, accb_ref, self1_ref, dis_ref, inv_ref, w2_ref,
              b1_ref, g2_ref, self2_ref):
    out1 = dis_ref[...] * (acca_ref[...] + accb_ref[...]) + self1_ref[...] \
        + b1_ref[...]
    z = jnp.maximum(out1, 0.0)
    h2 = jnp.dot(z, w2_ref[...], preferred_element_type=jnp.float32)
    g2_ref[...] = h2 * dis_ref[...]
    self2_ref[...] = h2 * inv_ref[...]


_tc2 = pl.pallas_call(
    _tc2_body,
    grid=(_GRID,),
    in_specs=[
        pl.BlockSpec((_BLK, D_HID), lambda i: (i, 0)),
        pl.BlockSpec((_BLK, D_HID), lambda i: (i, 0)),
        pl.BlockSpec((_BLK, D_HID), lambda i: (i, 0)),
        pl.BlockSpec((_BLK, 1), lambda i: (i, 0)),
        pl.BlockSpec((_BLK, 1), lambda i: (i, 0)),
        pl.BlockSpec((D_HID, N_CLS), lambda i: (0, 0)),
        pl.BlockSpec((1, D_HID), lambda i: (0, 0)),
    ],
    out_specs=[
        pl.BlockSpec((_BLK, N_CLS), lambda i: (i, 0)),
        pl.BlockSpec((_BLK, N_CLS), lambda i: (i, 0)),
    ],
    out_shape=[
        jax.ShapeDtypeStruct((N_PAD, N_CLS), jnp.float32),
        jax.ShapeDtypeStruct((N_PAD, N_CLS), jnp.float32),
    ],
)


def _tc3_body(acca_ref, accb_ref, self2_ref, dis_ref, b2_ref, out_ref):
    s = dis_ref[...] * (acca_ref[...] + accb_ref[...]) \
        + self2_ref[...] + b2_ref[...]
    m = jnp.max(s, axis=1, keepdims=True)
    e = s - m
    out_ref[...] = e - jnp.log(jnp.sum(jnp.exp(e), axis=1, keepdims=True))


_tc3 = pl.pallas_call(
    _tc3_body,
    grid=(_GRID,),
    in_specs=[
        pl.BlockSpec((_BLK, N_CLS), lambda i: (i, 0)),
        pl.BlockSpec((_BLK, N_CLS), lambda i: (i, 0)),
        pl.BlockSpec((_BLK, N_CLS), lambda i: (i, 0)),
        pl.BlockSpec((_BLK, 1), lambda i: (i, 0)),
        pl.BlockSpec((1, N_CLS), lambda i: (0, 0)),
    ],
    out_specs=pl.BlockSpec((_BLK, N_CLS), lambda i: (i, 0)),
    out_shape=jax.ShapeDtypeStruct((N, N_CLS), jnp.float32),
)


def kernel(x, edge_index, W1, b1, W2, b2):
    edges = jnp.pad(edge_index, ((0, 0), (0, E_PAD - E)),
                    constant_values=N_PAD - 1).reshape(2 * ROWS_E, 128)

    h1 = _mm1(x, W1)
    deg_a, deg_b = _deg_kernel(edges)
    g1, self1, dis, inv = _tc1(h1,
                               deg_a.reshape(N_PAD, 1),
                               deg_b.reshape(N_PAD, 1))
    acc1_a, acc1_b = _prop16(g1, edges)
    g2, self2 = _tc2(acc1_a, acc1_b, self1, dis, inv, W2,
                     b1.reshape(1, D_HID))
    acc2_a, acc2_b = _prop40(g2, edges)
    return _tc3(acc2_a, acc2_b, self2, dis, b2.reshape(1, N_CLS))


# speedup vs baseline: 48.1056x; 1.0148x over previous
import functools

import jax
import jax.numpy as jnp
from jax import lax
from jax.experimental import pallas as pl
from jax.experimental.pallas import tpu as pltpu
from jax.experimental.pallas import tpu_sc as plsc

N = 10000
E = 320000
D_IN = 128
D_HID = 16
N_CLS = 40

NSC = 2
NTILE = 16
NW = NSC * NTILE

N_PAD = 10240
ROWS_PER_TILE = N_PAD // NTILE
E_PAD = 4096 * 80
ROWS_E = E_PAD // 128
ROWS_E_TILE = ROWS_E // NW
CH = 128

_mesh = plsc.VectorSubcoreMesh(core_axis_name="c", subcore_axis_name="s")
_sc_params = pltpu.CompilerParams(use_tc_tiling_on_sc=False,
                                  needs_layout_passes=False)


@functools.partial(
    pl.kernel,
    out_type=(jax.ShapeDtypeStruct((N_PAD,), jnp.float32),
              jax.ShapeDtypeStruct((N_PAD,), jnp.float32)),
    mesh=_mesh,
    compiler_params=_sc_params,
    scratch_types=[
        pltpu.VMEM((ROWS_E_TILE, 128), jnp.int32),
        pltpu.VMEM((128,), jnp.float32),
        pltpu.VMEM((ROWS_PER_TILE,), jnp.float32),
        pltpu.VMEM_SHARED((N_PAD,), jnp.float32),
    ],
)
def _deg_kernel(edges_hbm, out0, out1, dst_v, ones_v, buf_v, deg_sh):
    cid = lax.axis_index("c")
    sid = lax.axis_index("s")
    wid = cid * NTILE + sid
    r0 = sid * ROWS_PER_TILE

    def fill_ones(i, carry):
        ones_v[pl.ds(i * 16, 16)] = jnp.full((16,), 1.0, jnp.float32)
        return carry
    lax.fori_loop(0, 128 // 16, fill_ones, 0)

    def fill_zero(i, carry):
        buf_v[pl.ds(i * 16, 16)] = jnp.zeros((16,), jnp.float32)
        return carry
    lax.fori_loop(0, ROWS_PER_TILE // 16, fill_zero, 0)

    pltpu.sync_copy(buf_v, deg_sh.at[pl.ds(r0, ROWS_PER_TILE)])
    pltpu.sync_copy(
        edges_hbm.at[pl.ds(ROWS_E + wid * ROWS_E_TILE, ROWS_E_TILE)], dst_v)
    plsc.subcore_barrier()

    def step(c, carry):
        pltpu.sync_copy(ones_v, deg_sh.at[dst_v.at[c]], add=True)
        return carry
    lax.fori_loop(0, ROWS_E_TILE, step, 0)

    plsc.subcore_barrier()
    pltpu.sync_copy(deg_sh.at[pl.ds(r0, ROWS_PER_TILE)], buf_v)

    @pl.when(cid == 0)
    def _():
        pltpu.sync_copy(buf_v, out0.at[pl.ds(r0, ROWS_PER_TILE)])

    @pl.when(cid == 1)
    def _():
        pltpu.sync_copy(buf_v, out1.at[pl.ds(r0, ROWS_PER_TILE)])


def _col_groups(d):
    groups = []
    c = 0
    while c + 16 <= d:
        groups.append((c, 0))
        c += 16
    if c < d:
        groups.append((d - 16, 16 - (d - c)))
    return groups


def _make_prop_kernel(d, g):
    G = g
    G128 = G * 128
    STEPS = ROWS_E_TILE // G
    PAIRS = STEPS // 2
    assert STEPS * G == ROWS_E_TILE and PAIRS * 2 == STEPS
    GRPS = _col_groups(d)
    NCHUNK = ROWS_PER_TILE // CH

    @functools.partial(
        pl.kernel,
        out_type=(jax.ShapeDtypeStruct((N_PAD, d), jnp.float32),
                  jax.ShapeDtypeStruct((N_PAD, d), jnp.float32)),
        mesh=_mesh,
        compiler_params=_sc_params,
        scratch_types=[
            pltpu.VMEM((ROWS_E_TILE, 128), jnp.int32),
            pltpu.VMEM((ROWS_E_TILE, 128), jnp.int32),
            pltpu.VMEM((2 * G128, d), jnp.float32),
            pltpu.VMEM((CH, d), jnp.float32),
            pltpu.VMEM((CH, d), jnp.float32),
            pltpu.VMEM((ROWS_PER_TILE,), jnp.float32),
            pltpu.VMEM((ROWS_PER_TILE,), jnp.float32),
            pltpu.VMEM((ROWS_PER_TILE,), jnp.float32),
            pltpu.VMEM_SHARED((N_PAD, d), jnp.float32),
            pltpu.VMEM_SHARED((N_PAD, d), jnp.float32),
            pltpu.SemaphoreType.DMA,
            pltpu.SemaphoreType.DMA,
            pltpu.SemaphoreType.DMA,
        ],
    )
    def prop(h_hbm, dega_hbm, degb_hbm, edges_hbm, out0, out1,
             src_v, dst_v, rows_v, cbuf, tbuf, dis_v, da_v, db_v,
             table_sh, acc_sh, gsem0, gsem1, ssem):
        cid = lax.axis_index("c")
        sid = lax.axis_index("s")
        wid = cid * NTILE + sid
        r0 = sid * ROWS_PER_TILE
        lanes = lax.broadcasted_iota(jnp.int32, (16,), 0)

        pltpu.sync_copy(dega_hbm.at[pl.ds(r0, ROWS_PER_TILE)], da_v)
        pltpu.sync_copy(degb_hbm.at[pl.ds(r0, ROWS_PER_TILE)], db_v)

        def disrow(i, carry):
            dv = da_v[pl.ds(i * 16, 16)] + db_v[pl.ds(i * 16, 16)] + 1.0
            bits = plsc.bitcast(dv, jnp.int32)
            y = plsc.bitcast(0x5F3759DF - lax.shift_right_logical(bits, 1),
                             jnp.float32)
            for _ in range(3):
                y = y * (1.5 - 0.5 * dv * y * y)
            dis_v[pl.ds(i * 16, 16)] = y
            return carry
        lax.fori_loop(0, ROWS_PER_TILE // 16, disrow, 0)

        def stage_chunk(z, carry):
            base = z * CH
            pltpu.sync_copy(h_hbm.at[pl.ds(r0 + base, CH)], cbuf)

            def scalerow(i, carry2):
                dv = plsc.load_gather(dis_v, [jnp.full((16,), base + i,
                                                       jnp.int32)])
                for (c0, lo) in GRPS:
                    f = dv if lo == 0 else jnp.where(lanes >= lo, dv, 1.0)
                    cbuf[i, pl.ds(c0, 16)] = cbuf[i, pl.ds(c0, 16)] * f
                return carry2
            lax.fori_loop(0, CH, scalerow, 0)
            pltpu.sync_copy(cbuf, table_sh.at[pl.ds(r0 + base, CH)])
            return carry
        lax.fori_loop(0, NCHUNK, stage_chunk, 0)

        def zrow(i, carry):
            for (c0, _) in GRPS:
                cbuf[i, pl.ds(c0, 16)] = jnp.zeros((16,), jnp.float32)
            return carry
        lax.fori_loop(0, CH, zrow, 0)
        for z in range(NCHUNK):
            pltpu.sync_copy(cbuf, acc_sh.at[pl.ds(r0 + z * CH, CH)])

        pltpu.sync_copy(
            edges_hbm.at[pl.ds(wid * ROWS_E_TILE, ROWS_E_TILE)], src_v)
        pltpu.sync_copy(
            edges_hbm.at[pl.ds(ROWS_E + wid * ROWS_E_TILE, ROWS_E_TILE)], dst_v)
        plsc.subcore_barrier()

        def fire_gathers(grp, off, sem):
            for j in range(G):
                pltpu.async_copy(table_sh.at[src_v.at[grp * G + j]],
                                 rows_v.at[pl.ds(off + j * 128, 128)], sem)

        def drain_gathers(grp, off, sem):
            for j in range(G):
                pltpu.make_async_copy(
                    table_sh.at[src_v.at[grp * G + j]],
                    rows_v.at[pl.ds(off + j * 128, 128)], sem).wait()

        def scatter_group(grp, off):
            cps = [
                pltpu.async_copy(rows_v.at[pl.ds(off + j * 128, 128)],
                                 acc_sh.at[dst_v.at[grp * G + j]], ssem,
                                 add=True)
                for j in range(G)
            ]
            for cp in cps:
                cp.wait()

        fire_gathers(0, 0, gsem0)

        def step(t, carry):
            fire_gathers(2 * t + 1, G128, gsem1)
            drain_gathers(2 * t, 0, gsem0)
            scatter_group(2 * t, 0)

            @pl.when(t + 1 < PAIRS)
            def _():
                fire_gathers(2 * t + 2, 0, gsem0)
            drain_gathers(2 * t + 1, G128, gsem1)
            scatter_group(2 * t + 1, G128)
            return carry
        lax.fori_loop(0, PAIRS, step, 0)

        plsc.subcore_barrier()

        def post_chunk(z, carry):
            base = z * CH
            pltpu.sync_copy(acc_sh.at[pl.ds(r0 + base, CH)], cbuf)
            pltpu.sync_copy(table_sh.at[pl.ds(r0 + base, CH)], tbuf)

            def postrow(i, carry2):
                dv = plsc.load_gather(dis_v, [jnp.full((16,), base + i,
                                                       jnp.int32)])
                for (c0, lo) in GRPS:
                    a = cbuf[i, pl.ds(c0, 16)]
                    t = tbuf[i, pl.ds(c0, 16)]
                    if lo == 0:
                        f, tg = dv, t
                    else:
                        f = jnp.where(lanes >= lo, dv, 1.0)
                        tg = jnp.where(lanes >= lo, t, 0.0)
                    tg = jnp.where(cid == 0, tg, jnp.zeros((16,), jnp.float32))
                    cbuf[i, pl.ds(c0, 16)] = (a + tg) * f
                return carry2
            lax.fori_loop(0, CH, postrow, 0)

            @pl.when(cid == 0)
            def _():
                pltpu.sync_copy(cbuf, out0.at[pl.ds(r0 + base, CH)])

            @pl.when(cid == 1)
            def _():
                pltpu.sync_copy(cbuf, out1.at[pl.ds(r0 + base, CH)])
            return carry
        lax.fori_loop(0, NCHUNK, post_chunk, 0)

    return prop


_prop16 = _make_prop_kernel(D_HID, 8)
_prop40 = _make_prop_kernel(N_CLS, 2)


def _mm1_body(x_ref, w1_ref, h_ref):
    h_ref[...] = jnp.dot(x_ref[...], w1_ref[...],
                         preferred_element_type=jnp.float32)


_mm1 = pl.pallas_call(
    _mm1_body,
    grid=(1,),
    in_specs=[
        pl.BlockSpec((N, D_IN), lambda i: (0, 0)),
        pl.BlockSpec((D_IN, D_HID), lambda i: (0, 0)),
    ],
    out_specs=pl.BlockSpec((N, D_HID), lambda i: (0, 0)),
    out_shape=jax.ShapeDtypeStruct((N_PAD, D_HID), jnp.float32),
)


def _tc2_body(resa_ref, resb_ref, w2_ref, b1_ref, h2_ref):
    z = jnp.maximum(resa_ref[...] + resb_ref[...] + b1_ref[...], 0.0)
    h2_ref[...] = jnp.dot(z, w2_ref[...], preferred_element_type=jnp.float32)


_tc2 = pl.pallas_call(
    _tc2_body,
    grid=(1,),
    in_specs=[
        pl.BlockSpec((N, D_HID), lambda i: (0, 0)),
        pl.BlockSpec((N, D_HID), lambda i: (0, 0)),
        pl.BlockSpec((D_HID, N_CLS), lambda i: (0, 0)),
        pl.BlockSpec((1, D_HID), lambda i: (0, 0)),
    ],
    out_specs=pl.BlockSpec((N, N_CLS), lambda i: (0, 0)),
    out_shape=jax.ShapeDtypeStruct((N_PAD, N_CLS), jnp.float32),
)


def _tc3_body(resa_ref, resb_ref, b2_ref, out_ref):
    s = resa_ref[...] + resb_ref[...] + b2_ref[...]
    m = jnp.max(s, axis=1, keepdims=True)
    e = s - m
    out_ref[...] = e - jnp.log(jnp.sum(jnp.exp(e), axis=1, keepdims=True))


_tc3 = pl.pallas_call(
    _tc3_body,
    grid=(1,),
    in_specs=[
        pl.BlockSpec((N, N_CLS), lambda i: (0, 0)),
        pl.BlockSpec((N, N_CLS), lambda i: (0, 0)),
        pl.BlockSpec((1, N_CLS), lambda i: (0, 0)),
    ],
    out_specs=pl.BlockSpec((N, N_CLS), lambda i: (0, 0)),
    out_shape=jax.ShapeDtypeStruct((N, N_CLS), jnp.float32),
)


def kernel(x, edge_index, W1, b1, W2, b2):
    edges = jnp.pad(edge_index, ((0, 0), (0, E_PAD - E)),
                    constant_values=N_PAD - 1).reshape(2 * ROWS_E, 128)

    h1 = _mm1(x, W1)
    deg_a, deg_b = _deg_kernel(edges)
    res1_a, res1_b = _prop16(h1, deg_a, deg_b, edges)
    h2 = _tc2(res1_a, res1_b, W2, b1.reshape(1, D_HID))
    res2_a, res2_b = _prop40(h2, deg_a, deg_b, edges)
    return _tc3(res2_a, res2_b, b2.reshape(1, N_CLS))
